# Initial kernel scaffold; baseline (speedup 1.0000x reference)
import jax
import jax.numpy as jnp
from jax.experimental import pallas as pl


def kernel(coo, x, W1, b1, W2, b2):
    raise NotImplementedError("write your pallas kernel here")


# speedup vs baseline: 14.2402x; 14.2402x over previous
import functools

import jax
import jax.numpy as jnp
from jax.experimental import pallas as pl
from jax.experimental.pallas import tpu as pltpu

K = 16
NPG = 100
BIG = 1 << 30


def _graph_kernel(posp_ref, x_ref, w1_ref, b1_ref, w2_ref, b2_ref, out_ref):
    posp = posp_ref[0]
    x = x_ref[0]

    g = jax.lax.dot_general(posp, posp, (((1,), (1,)), ((), ())),
                            preferred_element_type=jnp.float32)
    ri = jax.lax.broadcasted_iota(jnp.int32, (NPG, NPG), 0)
    ci = jax.lax.broadcasted_iota(jnp.int32, (NPG, NPG), 1)
    eye = (ri == ci)
    eyef = jnp.where(eye, 1.0, 0.0)
    sq_col = jnp.sum(g * eyef, axis=1, keepdims=True)
    sq_row = jnp.sum(g * eyef, axis=0, keepdims=True)
    d = sq_col + sq_row - 2.0 * g

    key = d.astype(jnp.int32) * 128 + ci
    key = jnp.where(eye, BIG, key)

    adj = eyef
    for _ in range(K):
        m = jnp.min(key, axis=1, keepdims=True)
        hit = key == m
        adj = adj + jnp.where(hit, 1.0, 0.0)
        key = jnp.where(hit, BIG, key)

    inv_deg = jnp.float32(1.0 / 17.0)
    xw1 = jnp.dot(x, w1_ref[0], preferred_element_type=jnp.float32)
    h = jnp.dot(adj, xw1, preferred_element_type=jnp.float32)
    h = jax.nn.relu(h * inv_deg + b1_ref[0])
    hw2 = jnp.dot(h, w2_ref[0], preferred_element_type=jnp.float32)
    o = jnp.dot(adj, hw2, preferred_element_type=jnp.float32)
    out_ref[0] = o * inv_deg + b2_ref[0]


@jax.jit
def kernel(coo, x, W1, b1, W2, b2):
    N = x.shape[0]
    B = N // NPG
    d_in = x.shape[1]
    d_hid = W1.shape[1]
    d_out = W2.shape[1]

    posp = jnp.zeros((B, NPG, 8), jnp.float32)
    posp = posp.at[:, :, :2].set(
        coo[:, :2].astype(jnp.float32).reshape(B, NPG, 2))
    x3 = x.reshape(B, NPG, d_in)

    out = pl.pallas_call(
        _graph_kernel,
        grid=(B,),
        in_specs=[
            pl.BlockSpec((1, NPG, 8), lambda b: (b, 0, 0)),
            pl.BlockSpec((1, NPG, d_in), lambda b: (b, 0, 0)),
            pl.BlockSpec((1, d_in, d_hid), lambda b: (0, 0, 0)),
            pl.BlockSpec((1, 1, d_hid), lambda b: (0, 0, 0)),
            pl.BlockSpec((1, d_hid, d_out), lambda b: (0, 0, 0)),
            pl.BlockSpec((1, 1, d_out), lambda b: (0, 0, 0)),
        ],
        out_specs=pl.BlockSpec((1, NPG, d_out), lambda b: (b, 0, 0)),
        out_shape=jax.ShapeDtypeStruct((B, NPG, d_out), jnp.float32),
    )(posp, x3, W1[None], b1[None, None], W2[None], b2[None, None])
    return out.reshape(N, d_out)


# speedup vs baseline: 23.5900x; 1.6566x over previous
import functools

import jax
import jax.numpy as jnp
from jax import lax
from jax.experimental import pallas as pl
from jax.experimental.pallas import tpu as pltpu
from jax.experimental.pallas import tpu_sc as plsc

K = 16
NPG = 100
NPAD = 112
NGRP = NPAD // 16
BIG = 1 << 30
INV_DEG = 1.0 / 17.0
NWORKERS = 32


def _mm1_kernel(x_ref, w_ref, o_ref):
    o_ref[...] = jnp.dot(x_ref[...], w_ref[...],
                         preferred_element_type=jnp.float32)


def _mm2_kernel(g_ref, w_ref, b_ref, o_ref):
    o_ref[...] = (jnp.dot(g_ref[...], w_ref[...],
                          preferred_element_type=jnp.float32) * INV_DEG
                  + b_ref[...])


def _sc_body(xs_hbm, ys_hbm, xw1_hbm, b1b_hbm, g2_hbm,
             xs_v, ys_v, xw1_v, h_v, g2_v, idxT_v, b1b_v):
    B = xs_hbm.shape[0]
    wid = lax.axis_index("s") * 2 + lax.axis_index("c")

    pltpu.sync_copy(b1b_hbm, b1b_v)
    lane = lax.iota(jnp.int32, 16)

    def _merge(a, b):
        ak, av = a
        bk, bv = b
        bk2 = lax.rev(bk, (0,))
        bv2 = lax.rev(bv, (0,))
        ta = ak <= bk2
        ck = jnp.where(ta, ak, bk2)
        cv = jnp.where(ta, av, bv2)
        return plsc.sort_key_val(ck, cv)

    def _process_graph(g):
        pltpu.sync_copy(xs_hbm.at[g], xs_v)
        pltpu.sync_copy(ys_hbm.at[g], ys_v)
        pltpu.sync_copy(xw1_hbm.at[g], xw1_v.at[pl.ds(0, NPG)])
        for n in range(K):
            idxT_v[n, pl.ds(NPG - 4, 16)] = jnp.zeros((16,), jnp.int32)

        def _node(i, _):
            ii = jnp.full((16,), i, jnp.int32)
            xi = plsc.load_gather(xs_v, [ii])
            yi = plsc.load_gather(ys_v, [ii])
            groups = []
            for j in range(NGRP):
                xg = xs_v[pl.ds(16 * j, 16)]
                yg = ys_v[pl.ds(16 * j, 16)]
                dx = xg - xi
                dy = yg - yi
                d = dx * dx + dy * dy
                jv = lane + (16 * j)
                key = d * 128 + jv
                if 16 * (j + 1) > NPG:
                    key = jnp.where(jv >= NPG, BIG, key)
                key = jnp.where(jv == i, BIG, key)
                groups.append(plsc.sort_key_val(key, jv))
            m01 = _merge(groups[0], groups[1])
            m23 = _merge(groups[2], groups[3])
            m45 = _merge(groups[4], groups[5])
            m0123 = _merge(m01, m23)
            m456 = _merge(m45, groups[6])
            _, va = _merge(m0123, m456)
            plsc.store_scatter(idxT_v, [lane, ii], va)
            return 0

        lax.fori_loop(0, NPG, _node, 0)

        def _agg(gi, src_v, dst_v, relu_bias):
            base = gi * 16
            nodes = lane + base
            accs = [plsc.load_gather(src_v, [nodes, jnp.full((16,), f, jnp.int32)])
                    for f in range(K)]
            for n in range(K):
                idx_n = idxT_v[n, pl.ds(base, 16)]
                for f in range(K):
                    accs[f] = accs[f] + plsc.load_gather(
                        src_v, [idx_n, jnp.full((16,), f, jnp.int32)])
            for f in range(K):
                v = accs[f]
                if relu_bias:
                    v = jnp.maximum(v * INV_DEG + b1b_v[f], 0.0)
                plsc.store_scatter(
                    dst_v, [nodes, jnp.full((16,), f, jnp.int32)], v)
            return 0

        lax.fori_loop(0, NGRP, lambda gi, c: _agg(gi, xw1_v, h_v, True), 0)
        lax.fori_loop(0, NGRP, lambda gi, c: _agg(gi, h_v, g2_v, False), 0)

        pltpu.sync_copy(g2_v.at[pl.ds(0, NPG)], g2_hbm.at[g])

    def _step(t, _):
        g = wid + NWORKERS * t

        @pl.when(g < B)
        def _():
            _process_graph(g)

        return 0

    lax.fori_loop(0, (B + NWORKERS - 1) // NWORKERS, _step, 0)


@jax.jit
def kernel(coo, x, W1, b1, W2, b2):
    N = x.shape[0]
    B = N // NPG
    d_in = x.shape[1]
    d_hid = W1.shape[1]
    d_out = W2.shape[1]

    blk = 2000
    xw1 = pl.pallas_call(
        _mm1_kernel,
        grid=(N // blk,),
        in_specs=[
            pl.BlockSpec((blk, d_in), lambda i: (i, 0)),
            pl.BlockSpec((d_in, d_hid), lambda i: (0, 0)),
        ],
        out_specs=pl.BlockSpec((blk, d_hid), lambda i: (i, 0)),
        out_shape=jax.ShapeDtypeStruct((N, d_hid), jnp.float32),
    )(x, W1)

    xs = jnp.zeros((B, NPAD), jnp.int32).at[:, :NPG].set(
        coo[:, 0].reshape(B, NPG))
    ys = jnp.zeros((B, NPAD), jnp.int32).at[:, :NPG].set(
        coo[:, 1].reshape(B, NPG))
    xw1r = xw1.reshape(B, NPG, d_hid)
    b1b = jnp.broadcast_to(b1[:, None], (d_hid, 16))

    mesh = plsc.VectorSubcoreMesh(core_axis_name="c", subcore_axis_name="s",
                                  num_cores=2, num_subcores=16)
    g2 = pl.kernel(
        _sc_body,
        out_type=jax.ShapeDtypeStruct((B, NPG, d_hid), jnp.float32),
        mesh=mesh,
        compiler_params=pltpu.CompilerParams(needs_layout_passes=False),
        scratch_types=[
            pltpu.VMEM((NPAD,), jnp.int32),
            pltpu.VMEM((NPAD,), jnp.int32),
            pltpu.VMEM((NPAD, d_hid), jnp.float32),
            pltpu.VMEM((NPAD, d_hid), jnp.float32),
            pltpu.VMEM((NPAD, d_hid), jnp.float32),
            pltpu.VMEM((K, NPAD), jnp.int32),
            pltpu.VMEM((d_hid, 16), jnp.float32),
        ],
    )(xs, ys, xw1r, b1b)

    out = pl.pallas_call(
        _mm2_kernel,
        grid=(N // blk,),
        in_specs=[
            pl.BlockSpec((blk, d_hid), lambda i: (i, 0)),
            pl.BlockSpec((d_hid, d_out), lambda i: (0, 0)),
            pl.BlockSpec((1, d_out), lambda i: (0, 0)),
        ],
        out_specs=pl.BlockSpec((blk, d_out), lambda i: (i, 0)),
        out_shape=jax.ShapeDtypeStruct((N, d_out), jnp.float32),
    )(g2.reshape(N, d_hid), W2, b2[None])
    return out
